# Initial kernel scaffold; baseline (speedup 1.0000x reference)
import jax
import jax.numpy as jnp
from jax.experimental import pallas as pl


def kernel(fatoms, fbonds, agraph, bgraph, scope, W_i, W_h, W_o_w, W_o_b):
    raise NotImplementedError("write your pallas kernel here")


# speedup vs baseline: 1.4898x; 1.4898x over previous
import functools

import jax
import jax.numpy as jnp
from jax import lax
from jax.experimental import pallas as pl
from jax.experimental.pallas import tpu as pltpu
from jax.experimental.pallas import tpu_sc as plsc

ATOM_FDIM = 39
BOND_FDIM = 11
BOND_IN = ATOM_FDIM + BOND_FDIM
MAX_NB = 6
HIDDEN = 128
DEPTH = 3
N_ATOMS = 100000
N_BONDS = 200000
N_MOLS = 500
MOL_LEN = 200

CH = 128
E_PAD = 204800
N_PAD = 102400
K_IN = 64
K_AT = 64


def _mm_relu_body(x_ref, w_ref, bin_ref, msg_ref):
    b = jnp.dot(x_ref[...], w_ref[...], preferred_element_type=jnp.float32)
    bin_ref[...] = b
    msg_ref[...] = jnp.maximum(b, 0.0)


def _bond_input(fbonds_pad, w_i_t):
    blk = 2048
    return pl.pallas_call(
        _mm_relu_body,
        grid=(E_PAD // blk,),
        in_specs=[
            pl.BlockSpec((blk, K_IN), lambda i: (i, 0)),
            pl.BlockSpec((K_IN, HIDDEN), lambda i: (0, 0)),
        ],
        out_specs=[
            pl.BlockSpec((blk, HIDDEN), lambda i: (i, 0)),
            pl.BlockSpec((blk, HIDDEN), lambda i: (i, 0)),
        ],
        out_shape=[
            jax.ShapeDtypeStruct((E_PAD, HIDDEN), jnp.float32),
            jax.ShapeDtypeStruct((E_PAD, HIDDEN), jnp.float32),
        ],
    )(fbonds_pad, w_i_t)


def _update_body(nei_ref, bin_ref, w_ref, out_ref):
    acc = jnp.dot(nei_ref[...], w_ref[...], preferred_element_type=jnp.float32)
    out_ref[...] = jnp.maximum(bin_ref[...] + acc, 0.0)


def _message_update(nei, binput, w_h_t):
    blk = 2048
    return pl.pallas_call(
        _update_body,
        grid=(E_PAD // blk,),
        in_specs=[
            pl.BlockSpec((blk, HIDDEN), lambda i: (i, 0)),
            pl.BlockSpec((blk, HIDDEN), lambda i: (i, 0)),
            pl.BlockSpec((HIDDEN, HIDDEN), lambda i: (0, 0)),
        ],
        out_specs=pl.BlockSpec((blk, HIDDEN), lambda i: (i, 0)),
        out_shape=jax.ShapeDtypeStruct((E_PAD, HIDDEN), jnp.float32),
    )(nei, binput, w_h_t)


def _atom_out_body(fa_ref, as_ref, wa_ref, wh_ref, b_ref, out_ref):
    acc = jnp.dot(fa_ref[...], wa_ref[...], preferred_element_type=jnp.float32)
    acc = acc + jnp.dot(as_ref[...], wh_ref[...], preferred_element_type=jnp.float32)
    out_ref[...] = jnp.maximum(acc + b_ref[...], 0.0)


def _atom_output(fatoms_pad, asum, w_oa_t, w_oh_t, bias):
    blk = 800
    return pl.pallas_call(
        _atom_out_body,
        grid=(N_ATOMS // blk,),
        in_specs=[
            pl.BlockSpec((blk, K_AT), lambda i: (i, 0)),
            pl.BlockSpec((blk, HIDDEN), lambda i: (i, 0)),
            pl.BlockSpec((K_AT, HIDDEN), lambda i: (0, 0)),
            pl.BlockSpec((HIDDEN, HIDDEN), lambda i: (0, 0)),
            pl.BlockSpec((1, HIDDEN), lambda i: (0, 0)),
        ],
        out_specs=pl.BlockSpec((blk, HIDDEN), lambda i: (i, 0)),
        out_shape=jax.ShapeDtypeStruct((N_ATOMS, HIDDEN), jnp.float32),
    )(fatoms_pad, asum, w_oa_t, w_oh_t, bias)


def _make_gather_sum(n_rows_pad):
    info = plsc.get_sparse_core_info()
    nc, ns = info.num_cores, info.num_subcores
    nw = nc * ns
    rpw = n_rows_pad // nw
    n_chunks = rpw // CH
    mesh = plsc.VectorSubcoreMesh(core_axis_name="c", subcore_axis_name="s")

    @functools.partial(
        pl.kernel,
        mesh=mesh,
        out_type=jax.ShapeDtypeStruct((n_rows_pad, HIDDEN), jnp.float32),
        scratch_types=[
            pltpu.VMEM((MAX_NB, CH), jnp.int32),
            pltpu.VMEM((MAX_NB, CH, HIDDEN), jnp.float32),
            pltpu.VMEM((CH, HIDDEN), jnp.float32),
            pltpu.SemaphoreType.DMA,
        ],
    )
    def gsum(table_hbm, idx_hbm, out_hbm, idx_v, rows_v, acc_v, sem):
        wid = lax.axis_index("s") * nc + lax.axis_index("c")

        def chunk_body(c, carry):
            g = wid * n_chunks + c
            pltpu.sync_copy(idx_hbm.at[g], idx_v)
            copies = [
                pltpu.async_copy(table_hbm.at[idx_v.at[j]], rows_v.at[j], sem)
                for j in range(MAX_NB)
            ]
            for cp in copies:
                cp.wait()

            def row_body(r, carry2):
                for gc in range(HIDDEN // 16):
                    sl = pl.ds(gc * 16, 16)
                    s = rows_v[0, r, sl]
                    for j in range(1, MAX_NB):
                        s = s + rows_v[j, r, sl]
                    acc_v[r, sl] = s
                return carry2

            lax.fori_loop(0, CH, row_body, 0, unroll=False)
            pltpu.sync_copy(acc_v, out_hbm.at[pl.ds(g * CH, CH)])
            return carry

        lax.fori_loop(0, n_chunks, chunk_body, 0, unroll=False)

    return gsum


def kernel(fatoms, fbonds, agraph, bgraph, scope, W_i, W_h, W_o_w, W_o_b):
    f32 = jnp.float32
    fbonds_pad = jnp.zeros((E_PAD, K_IN), f32).at[:N_BONDS, :BOND_IN].set(fbonds)
    w_i_t = jnp.zeros((K_IN, HIDDEN), f32).at[:BOND_IN].set(W_i.T)
    w_h_t = W_h.T
    fatoms_pad = jnp.zeros((N_ATOMS, K_AT), f32).at[:, :ATOM_FDIM].set(fatoms)
    w_oa_t = jnp.zeros((K_AT, HIDDEN), f32).at[:ATOM_FDIM].set(W_o_w[:, :ATOM_FDIM].T)
    w_oh_t = W_o_w[:, ATOM_FDIM:].T
    bias = W_o_b.reshape(1, HIDDEN)

    bg = jnp.pad(bgraph.astype(jnp.int32), ((0, E_PAD - N_BONDS), (0, 0)))
    bg_chunks = bg.reshape(E_PAD // CH, CH, MAX_NB).transpose(0, 2, 1)
    ag = jnp.pad(agraph.astype(jnp.int32), ((0, N_PAD - N_ATOMS), (0, 0)))
    ag_chunks = ag.reshape(N_PAD // CH, CH, MAX_NB).transpose(0, 2, 1)

    binput, message = _bond_input(fbonds_pad, w_i_t)

    bond_gsum = _make_gather_sum(E_PAD)
    for _ in range(DEPTH - 1):
        nei = bond_gsum(message, bg_chunks)
        message = _message_update(nei, binput, w_h_t)

    atom_gsum = _make_gather_sum(N_PAD)
    asum = atom_gsum(message, ag_chunks)

    atom_hiddens = _atom_output(fatoms_pad, asum, w_oa_t, w_oh_t, bias)

    mol = atom_hiddens.reshape(N_MOLS, MOL_LEN, HIDDEN)
    lengths = scope[:, 1].astype(jnp.int32)
    mask = jnp.arange(MOL_LEN, dtype=jnp.int32)[None, :] < lengths[:, None]
    return jnp.where(mask[:, :, None], mol, jnp.zeros((), f32))


# speedup vs baseline: 1.6529x; 1.1094x over previous
import functools

import jax
import jax.numpy as jnp
from jax import lax
from jax.experimental import pallas as pl
from jax.experimental.pallas import tpu as pltpu
from jax.experimental.pallas import tpu_sc as plsc

ATOM_FDIM = 39
BOND_FDIM = 11
BOND_IN = ATOM_FDIM + BOND_FDIM
MAX_NB = 6
HIDDEN = 128
DEPTH = 3
N_ATOMS = 100000
N_BONDS = 200000
N_MOLS = 500
MOL_LEN = 200

CH = 64
E_PAD = 204800
N_PAD = 102400
K_IN = 64
K_AT = 64


def _mm_relu_body(x_ref, w_ref, bin_ref, msg_ref):
    b = jnp.dot(x_ref[...], w_ref[...], preferred_element_type=jnp.float32)
    bin_ref[...] = b
    msg_ref[...] = jnp.maximum(b, 0.0)


def _bond_input(fbonds_pad, w_i_t):
    blk = 2048
    return pl.pallas_call(
        _mm_relu_body,
        grid=(E_PAD // blk,),
        in_specs=[
            pl.BlockSpec((blk, K_IN), lambda i: (i, 0)),
            pl.BlockSpec((K_IN, HIDDEN), lambda i: (0, 0)),
        ],
        out_specs=[
            pl.BlockSpec((blk, HIDDEN), lambda i: (i, 0)),
            pl.BlockSpec((blk, HIDDEN), lambda i: (i, 0)),
        ],
        out_shape=[
            jax.ShapeDtypeStruct((E_PAD, HIDDEN), jnp.float32),
            jax.ShapeDtypeStruct((E_PAD, HIDDEN), jnp.float32),
        ],
    )(fbonds_pad, w_i_t)


def _update_body(nei_ref, bin_ref, w_ref, out_ref):
    acc = jnp.dot(nei_ref[...], w_ref[...], preferred_element_type=jnp.float32)
    out_ref[...] = jnp.maximum(bin_ref[...] + acc, 0.0)


def _message_update(nei, binput, w_h_t):
    blk = 2048
    return pl.pallas_call(
        _update_body,
        grid=(E_PAD // blk,),
        in_specs=[
            pl.BlockSpec((blk, HIDDEN), lambda i: (i, 0)),
            pl.BlockSpec((blk, HIDDEN), lambda i: (i, 0)),
            pl.BlockSpec((HIDDEN, HIDDEN), lambda i: (0, 0)),
        ],
        out_specs=pl.BlockSpec((blk, HIDDEN), lambda i: (i, 0)),
        out_shape=jax.ShapeDtypeStruct((E_PAD, HIDDEN), jnp.float32),
    )(nei, binput, w_h_t)


def _atom_out_body(fa_ref, as_ref, wa_ref, wh_ref, b_ref, out_ref):
    acc = jnp.dot(fa_ref[...], wa_ref[...], preferred_element_type=jnp.float32)
    acc = acc + jnp.dot(as_ref[...], wh_ref[...], preferred_element_type=jnp.float32)
    out_ref[...] = jnp.maximum(acc + b_ref[...], 0.0)


def _atom_output(fatoms_pad, asum, w_oa_t, w_oh_t, bias):
    blk = 800
    return pl.pallas_call(
        _atom_out_body,
        grid=(N_ATOMS // blk,),
        in_specs=[
            pl.BlockSpec((blk, K_AT), lambda i: (i, 0)),
            pl.BlockSpec((blk, HIDDEN), lambda i: (i, 0)),
            pl.BlockSpec((K_AT, HIDDEN), lambda i: (0, 0)),
            pl.BlockSpec((HIDDEN, HIDDEN), lambda i: (0, 0)),
            pl.BlockSpec((1, HIDDEN), lambda i: (0, 0)),
        ],
        out_specs=pl.BlockSpec((blk, HIDDEN), lambda i: (i, 0)),
        out_shape=jax.ShapeDtypeStruct((N_ATOMS, HIDDEN), jnp.float32),
    )(fatoms_pad, asum, w_oa_t, w_oh_t, bias)


def _make_gather_sum(n_rows_pad):
    info = plsc.get_sparse_core_info()
    nc, ns = info.num_cores, info.num_subcores
    nw = nc * ns
    rpw = n_rows_pad // nw
    n_chunks = rpw // CH
    mesh = plsc.VectorSubcoreMesh(core_axis_name="c", subcore_axis_name="s")

    @functools.partial(
        pl.kernel,
        mesh=mesh,
        out_type=jax.ShapeDtypeStruct((n_rows_pad, HIDDEN), jnp.float32),
        scratch_types=[
            pltpu.VMEM((2, MAX_NB, CH), jnp.int32),
            pltpu.VMEM((2, MAX_NB, CH, HIDDEN), jnp.float32),
            pltpu.VMEM((CH, HIDDEN), jnp.float32),
            pltpu.SemaphoreType.DMA,
            pltpu.SemaphoreType.DMA,
            pltpu.SemaphoreType.DMA,
            pltpu.SemaphoreType.DMA,
        ],
    )
    def gsum(table_hbm, idx_hbm, out_hbm, idx_v, rows_v, acc_v,
             gsem0, gsem1, isem0, isem1):
        wid = lax.axis_index("s") * nc + lax.axis_index("c")
        g0 = wid * n_chunks
        gsems = (gsem0, gsem1)
        isems = (isem0, isem1)

        def fire_gathers(b, sem):
            for j in range(MAX_NB):
                pltpu.async_copy(table_hbm.at[idx_v.at[b, j]], rows_v.at[b, j], sem)

        def drain_gathers(b, sem):
            for j in range(MAX_NB):
                pltpu.make_async_copy(
                    table_hbm.at[idx_v.at[b, j]], rows_v.at[b, j], sem).wait()

        pltpu.sync_copy(idx_hbm.at[g0], idx_v.at[0])
        fire_gathers(0, gsems[0])

        def super_body(i, carry):
            for b in range(2):
                c = 2 * i + b
                nb = 1 - b

                @pl.when(c + 1 < n_chunks)
                def _():
                    pltpu.async_copy(idx_hbm.at[g0 + c + 1], idx_v.at[nb],
                                     isems[nb])

                drain_gathers(b, gsems[b])

                @pl.when(c + 1 < n_chunks)
                def _():
                    pltpu.make_async_copy(idx_hbm.at[g0 + c + 1],
                                          idx_v.at[nb], isems[nb]).wait()
                    fire_gathers(nb, gsems[nb])

                @plsc.parallel_loop(0, CH, unroll=4)
                def _(r):
                    for gc in range(HIDDEN // 16):
                        sl = pl.ds(gc * 16, 16)
                        s = rows_v[b, 0, r, sl]
                        for j in range(1, MAX_NB):
                            s = s + rows_v[b, j, r, sl]
                        acc_v[r, sl] = s

                pltpu.sync_copy(acc_v, out_hbm.at[pl.ds((g0 + c) * CH, CH)])
            return carry

        lax.fori_loop(0, n_chunks // 2, super_body, 0, unroll=False)

    return gsum


def kernel(fatoms, fbonds, agraph, bgraph, scope, W_i, W_h, W_o_w, W_o_b):
    f32 = jnp.float32
    fbonds_pad = jnp.zeros((E_PAD, K_IN), f32).at[:N_BONDS, :BOND_IN].set(fbonds)
    w_i_t = jnp.zeros((K_IN, HIDDEN), f32).at[:BOND_IN].set(W_i.T)
    w_h_t = W_h.T
    fatoms_pad = jnp.zeros((N_ATOMS, K_AT), f32).at[:, :ATOM_FDIM].set(fatoms)
    w_oa_t = jnp.zeros((K_AT, HIDDEN), f32).at[:ATOM_FDIM].set(W_o_w[:, :ATOM_FDIM].T)
    w_oh_t = W_o_w[:, ATOM_FDIM:].T
    bias = W_o_b.reshape(1, HIDDEN)

    bg = jnp.pad(bgraph.astype(jnp.int32), ((0, E_PAD - N_BONDS), (0, 0)))
    bg_chunks = bg.reshape(E_PAD // CH, CH, MAX_NB).transpose(0, 2, 1)
    ag = jnp.pad(agraph.astype(jnp.int32), ((0, N_PAD - N_ATOMS), (0, 0)))
    ag_chunks = ag.reshape(N_PAD // CH, CH, MAX_NB).transpose(0, 2, 1)

    binput, message = _bond_input(fbonds_pad, w_i_t)

    bond_gsum = _make_gather_sum(E_PAD)
    for _ in range(DEPTH - 1):
        nei = bond_gsum(message, bg_chunks)
        message = _message_update(nei, binput, w_h_t)

    atom_gsum = _make_gather_sum(N_PAD)
    asum = atom_gsum(message, ag_chunks)

    atom_hiddens = _atom_output(fatoms_pad, asum, w_oa_t, w_oh_t, bias)

    mol = atom_hiddens.reshape(N_MOLS, MOL_LEN, HIDDEN)
    lengths = scope[:, 1].astype(jnp.int32)
    mask = jnp.arange(MOL_LEN, dtype=jnp.int32)[None, :] < lengths[:, None]
    return jnp.where(mask[:, :, None], mol, jnp.zeros((), f32))
